# baseline (speedup 1.0000x reference)
import functools

import jax
import jax.numpy as jnp
from jax import lax
from jax.experimental import pallas as pl
from jax.experimental.pallas import tpu as pltpu
from jax.experimental.pallas import tpu_sc as plsc

VOCAB = 100000
D = 128
CTX = 200

_CH = 2
_CHN = CTX // _CH


def _sc_gather_sum(idx2, emb):
    mesh = plsc.VectorSubcoreMesh(core_axis_name="c", subcore_axis_name="s")

    @functools.partial(
        pl.kernel,
        out_type=jax.ShapeDtypeStruct((D,), jnp.float32),
        mesh=mesh,
        scratch_types=[
            pltpu.VMEM((_CH, _CHN), jnp.int32),
            pltpu.VMEM((_CH, _CHN, D), jnp.float32),
            pltpu.VMEM((D,), jnp.float32),
            pltpu.SemaphoreType.DMA,
        ],
    )
    def k(idx_hbm, emb_hbm, out_hbm, idx_v, rows_v, acc_v, sem):
        wid = lax.axis_index("s") * 2 + lax.axis_index("c")

        @pl.when(wid == 0)
        def _():
            pltpu.sync_copy(idx_hbm, idx_v)
            cps = [
                pltpu.async_copy(emb_hbm.at[idx_v.at[c]], rows_v.at[c], sem)
                for c in range(_CH)
            ]
            for cp in cps:
                cp.wait()

            def body(r, accs):
                new = []
                for d in range(D // 16):
                    a = accs[d]
                    for c in range(_CH):
                        a = a + rows_v[c, r, pl.ds(d * 16, 16)]
                    new.append(a)
                return tuple(new)

            accs = tuple(jnp.zeros((16,), jnp.float32) for _ in range(D // 16))
            accs = lax.fori_loop(0, _CHN, body, accs)
            for d in range(D // 16):
                acc_v[pl.ds(d * 16, 16)] = accs[d]
            pltpu.sync_copy(acc_v, out_hbm)

    return k(idx2, emb)


_BLK = 4096
_NB = (VOCAB + _BLK - 1) // _BLK


def _tc_logits_body(e_ref, w1_ref, b1_ref, w2_ref, b2_ref,
                    out_ref, logz_ref, h_ref, m_ref, s_ref):
    i = pl.program_id(0)

    @pl.when(i == 0)
    def _():
        h = jnp.dot(e_ref[...], w1_ref[...].T,
                    preferred_element_type=jnp.float32) + b1_ref[...]
        h_ref[...] = jnp.maximum(h, 0.0)
        m_ref[0] = -jnp.inf
        s_ref[0] = 0.0

    logits = lax.dot_general(
        h_ref[...], w2_ref[...], (((1,), (1,)), ((), ())),
        preferred_element_type=jnp.float32) + b2_ref[...]
    cols = i * _BLK + lax.broadcasted_iota(jnp.int32, (1, _BLK), 1)
    logits = jnp.where(cols < VOCAB, logits, -jnp.inf)
    out_ref[...] = logits

    bm = jnp.max(logits)
    m_old = m_ref[0]
    m_new = jnp.maximum(m_old, bm)
    s_ref[0] = s_ref[0] * jnp.exp(m_old - m_new) + jnp.sum(jnp.exp(logits - m_new))
    m_ref[0] = m_new

    @pl.when(i == _NB - 1)
    def _():
        logz_ref[...] = jnp.full((1, D), m_ref[0] + jnp.log(s_ref[0]),
                                 jnp.float32)


def _tc_logits(e, W1, b1, W2, b2r):
    return pl.pallas_call(
        _tc_logits_body,
        grid=(_NB,),
        in_specs=[
            pl.BlockSpec((1, D), lambda i: (0, 0)),
            pl.BlockSpec((D, D), lambda i: (0, 0)),
            pl.BlockSpec((1, D), lambda i: (0, 0)),
            pl.BlockSpec((_BLK, D), lambda i: (i, 0)),
            pl.BlockSpec((1, _BLK), lambda i: (0, i)),
        ],
        out_specs=[
            pl.BlockSpec((1, _BLK), lambda i: (0, i)),
            pl.BlockSpec((1, D), lambda i: (0, 0)),
        ],
        out_shape=[
            jax.ShapeDtypeStruct((1, VOCAB), jnp.float32),
            jax.ShapeDtypeStruct((1, D), jnp.float32),
        ],
        scratch_shapes=[
            pltpu.VMEM((1, D), jnp.float32),
            pltpu.SMEM((1,), jnp.float32),
            pltpu.SMEM((1,), jnp.float32),
        ],
    )(e, W1, b1, W2, b2r)


_BLK2 = 8192
_NB2 = (VOCAB + _BLK2 - 1) // _BLK2


def _tc_norm_body(logits_ref, logz_ref, out_ref):
    out_ref[...] = logits_ref[...] - logz_ref[0, 0]


def _tc_norm(logits, logz):
    return pl.pallas_call(
        _tc_norm_body,
        grid=(_NB2,),
        in_specs=[
            pl.BlockSpec((1, _BLK2), lambda i: (0, i)),
            pl.BlockSpec((1, D), lambda i: (0, 0)),
        ],
        out_specs=pl.BlockSpec((1, _BLK2), lambda i: (0, i)),
        out_shape=jax.ShapeDtypeStruct((1, VOCAB), jnp.float32),
    )(logits, logz)


def kernel(inputs, emb, W1, b1, W2, b2):
    idx2 = inputs.astype(jnp.int32).reshape(_CH, _CHN)
    e = _sc_gather_sum(idx2, emb).reshape(1, D)
    logits, logz = _tc_logits(e, W1, b1.reshape(1, D), W2, b2.reshape(1, VOCAB))
    return _tc_norm(logits, logz)
